# baseline (speedup 1.0000x reference)
import functools

import jax
import jax.numpy as jnp
from jax import lax
from jax.experimental import pallas as pl
from jax.experimental.pallas import tpu as pltpu
from jax.experimental.pallas import tpu_sc as plsc

EPS = 1e-12


def _sc_gather(ids_flat, contig_table, chunk):
    n = ids_flat.shape[0]
    vocab, hidden = contig_table.shape
    nch = n // chunk
    info = plsc.get_sparse_core_info()
    nw = info.num_cores * info.num_subcores
    iters = (nch + nw - 1) // nw
    mesh = plsc.VectorSubcoreMesh(core_axis_name="c", subcore_axis_name="s")

    @functools.partial(
        pl.kernel, mesh=mesh,
        out_type=jax.ShapeDtypeStruct((n, hidden), jnp.float32),
        scratch_types=[
            pltpu.VMEM((chunk,), jnp.int32),
            pltpu.VMEM((chunk, hidden), jnp.float32),
            pltpu.SemaphoreType.DMA,
        ],
    )
    def sck(ids_hbm, table_hbm, out_hbm, idx_v, rows_v, sem):
        wid = lax.axis_index("s") * info.num_cores + lax.axis_index("c")

        def do_chunk(cid):
            @pl.when(cid < nch)
            def _():
                base = cid * chunk
                pltpu.sync_copy(ids_hbm.at[pl.ds(base, chunk)], idx_v)
                pltpu.async_copy(table_hbm.at[idx_v], rows_v, sem).wait()
                pltpu.sync_copy(rows_v, out_hbm.at[pl.ds(base, chunk)])

        for c in range(iters):
            do_chunk(c * nw + wid)

    return sck(ids_flat, contig_table)


def _body(mask_ref, pe_ref, g_ref, w_ref, b_ref, me_ref, gam_ref,
          bt_ref, out_ref, *, block_rows):
    ji = pl.program_id(0)
    bi = pl.program_id(1)
    bsz, seq = mask_ref.shape

    mask_all = mask_ref[...]
    rowsel = lax.broadcasted_iota(jnp.int32, (bsz, 1), 0) == bi
    m_row = jnp.sum(jnp.where(rowsel, mask_all, 0), axis=0,
                    keepdims=True).astype(jnp.float32)
    m_win = m_row[:, 0:block_rows]
    for k in range(1, seq // block_rows):
        m_win = jnp.where(ji == k, m_row[:, k * block_rows:(k + 1) * block_rows],
                          m_win)
    ones11 = jnp.ones((1, 1), jnp.float32)
    m_col = lax.dot_general(
        m_win, ones11, (((0,), (0,)), ((), ())),
        precision=lax.Precision.HIGHEST,
        preferred_element_type=jnp.float32)

    x = lax.dot_general(
        pe_ref[0].astype(jnp.bfloat16), w_ref[...].astype(jnp.bfloat16),
        (((1,), (1,)), ((), ())),
        preferred_element_type=jnp.float32)
    x = x + b_ref[...]
    x = jnp.where(m_col > 0.5, me_ref[...], x)
    x = x + g_ref[:, 0:x.shape[1]]

    mean = jnp.mean(x, axis=1, keepdims=True)
    xc = x - mean
    var = jnp.mean(xc * xc, axis=1, keepdims=True)
    out_ref[0] = xc * lax.rsqrt(var + EPS) * gam_ref[...] + bt_ref[...]


@jax.jit
def kernel(protein_embeddings, contig_ids, mlm_mask, W, b, mask_embed,
           contig_table, ln_gamma, ln_beta):
    bsz, seq, hidden = protein_embeddings.shape
    n = bsz * seq

    block_rows = seq
    for cand in range(min(1200, seq), 7, -1):
        if seq % cand == 0 and cand % 8 == 0:
            block_rows = cand
            break
    nblk = seq // block_rows

    chunk = 120
    while n % chunk:
        chunk -= 8
    ids_flat = contig_ids.astype(jnp.int32).reshape(n)
    hpad = ((hidden + 127) // 128) * 128
    tbl_pad = jnp.zeros((contig_table.shape[0], hpad), jnp.float32)
    tbl_pad = tbl_pad.at[:, :hidden].set(contig_table)
    g = _sc_gather(ids_flat, tbl_pad, chunk)

    mask2d = mlm_mask.astype(jnp.int32)

    row = lambda v: v.reshape(1, hidden)

    out = pl.pallas_call(
        functools.partial(_body, block_rows=block_rows),
        grid=(nblk, bsz),
        in_specs=[
            pl.BlockSpec((bsz, seq), lambda j, i: (0, 0)),
            pl.BlockSpec((1, block_rows, hidden), lambda j, i: (i, j, 0)),
            pl.BlockSpec((block_rows, g.shape[1]),
                         lambda j, i: (i * (seq // block_rows) + j, 0)),
            pl.BlockSpec((hidden, hidden), lambda j, i: (0, 0)),
            pl.BlockSpec((1, hidden), lambda j, i: (0, 0)),
            pl.BlockSpec((1, hidden), lambda j, i: (0, 0)),
            pl.BlockSpec((1, hidden), lambda j, i: (0, 0)),
            pl.BlockSpec((1, hidden), lambda j, i: (0, 0)),
        ],
        out_specs=pl.BlockSpec((1, block_rows, hidden), lambda j, i: (i, j, 0)),
        out_shape=jax.ShapeDtypeStruct((bsz, seq, hidden), jnp.float32),
        compiler_params=pltpu.CompilerParams(
            dimension_semantics=("arbitrary", "arbitrary"),
        ),
    )(mask2d, protein_embeddings, g, W, row(b), row(mask_embed),
      row(ln_gamma), row(ln_beta))

    return out


# speedup vs baseline: 1.0817x; 1.0817x over previous
import functools

import jax
import jax.numpy as jnp
from jax import lax
from jax.experimental import pallas as pl
from jax.experimental.pallas import tpu as pltpu
from jax.experimental.pallas import tpu_sc as plsc

EPS = 1e-12


def _sc_gather(ids_flat, contig_table, chunk):
    n = ids_flat.shape[0]
    vocab, hidden = contig_table.shape
    nch = n // chunk
    info = plsc.get_sparse_core_info()
    nw = info.num_cores * info.num_subcores
    iters = (nch + nw - 1) // nw
    mesh = plsc.VectorSubcoreMesh(core_axis_name="c", subcore_axis_name="s")

    @functools.partial(
        pl.kernel, mesh=mesh,
        out_type=jax.ShapeDtypeStruct((n, hidden), jnp.int32),
        scratch_types=[
            pltpu.VMEM((chunk,), jnp.int32),
            pltpu.VMEM((chunk, hidden), jnp.int32),
            pltpu.SemaphoreType.DMA,
        ],
    )
    def sck(ids_hbm, table_hbm, out_hbm, idx_v, rows_v, sem):
        wid = lax.axis_index("s") * info.num_cores + lax.axis_index("c")

        def do_chunk(cid):
            @pl.when(cid < nch)
            def _():
                base = cid * chunk
                pltpu.sync_copy(ids_hbm.at[pl.ds(base, chunk)], idx_v)
                pltpu.async_copy(table_hbm.at[idx_v], rows_v, sem).wait()
                pltpu.sync_copy(rows_v, out_hbm.at[pl.ds(base, chunk)])

        for c in range(iters):
            do_chunk(c * nw + wid)

    return sck(ids_flat, contig_table)


def _body(mask_ref, pe_ref, g_ref, w_ref, b_ref, me_ref, gam_ref,
          bt_ref, out_ref, *, block_rows):
    ji = pl.program_id(0)
    bi = pl.program_id(1)
    bsz, seq = mask_ref.shape

    mask_all = mask_ref[...]
    rowsel = lax.broadcasted_iota(jnp.int32, (bsz, 1), 0) == bi
    m_row = jnp.sum(jnp.where(rowsel, mask_all, 0), axis=0,
                    keepdims=True).astype(jnp.float32)
    m_win = m_row[:, 0:block_rows]
    for k in range(1, seq // block_rows):
        m_win = jnp.where(ji == k, m_row[:, k * block_rows:(k + 1) * block_rows],
                          m_win)
    ones11 = jnp.ones((1, 1), jnp.float32)
    m_col = lax.dot_general(
        m_win, ones11, (((0,), (0,)), ((), ())),
        precision=lax.Precision.HIGHEST,
        preferred_element_type=jnp.float32)

    x = lax.dot_general(
        pe_ref[0].astype(jnp.bfloat16), w_ref[...].astype(jnp.bfloat16),
        (((1,), (1,)), ((), ())),
        preferred_element_type=jnp.float32)
    x = x + b_ref[...]
    x = jnp.where(m_col > 0.5, me_ref[...], x)
    w_pack = g_ref[...]
    lo = lax.bitcast_convert_type(
        lax.shift_left(w_pack, jnp.int32(16)), jnp.float32)
    hi = lax.bitcast_convert_type(
        lax.bitwise_and(w_pack, jnp.int32(-65536)), jnp.float32)
    g1024 = jnp.concatenate([lo, hi], axis=1)
    x = x + g1024[:, 0:x.shape[1]]

    mean = jnp.mean(x, axis=1, keepdims=True)
    xc = x - mean
    var = jnp.mean(xc * xc, axis=1, keepdims=True)
    out_ref[0] = xc * lax.rsqrt(var + EPS) * gam_ref[...] + bt_ref[...]


@jax.jit
def kernel(protein_embeddings, contig_ids, mlm_mask, W, b, mask_embed,
           contig_table, ln_gamma, ln_beta):
    bsz, seq, hidden = protein_embeddings.shape
    n = bsz * seq

    block_rows = seq
    for cand in range(min(1200, seq), 7, -1):
        if seq % cand == 0 and cand % 8 == 0:
            block_rows = cand
            break
    nblk = seq // block_rows

    chunk = 120
    while n % chunk:
        chunk -= 8
    ids_flat = contig_ids.astype(jnp.int32).reshape(n)
    hpad = ((hidden + 127) // 128) * 128
    half = hpad // 2
    bits = lax.bitcast_convert_type(
        contig_table.astype(jnp.bfloat16), jnp.uint16).astype(jnp.uint32)
    bits_pad = jnp.zeros((contig_table.shape[0], hpad), jnp.uint32)
    bits_pad = bits_pad.at[:, :hidden].set(bits)
    tbl_pack = (bits_pad[:, :half]
                | (bits_pad[:, half:] << 16)).astype(jnp.int32)
    g = _sc_gather(ids_flat, tbl_pack, chunk)

    mask2d = mlm_mask.astype(jnp.int32)

    row = lambda v: v.reshape(1, hidden)

    out = pl.pallas_call(
        functools.partial(_body, block_rows=block_rows),
        grid=(nblk, bsz),
        in_specs=[
            pl.BlockSpec((bsz, seq), lambda j, i: (0, 0)),
            pl.BlockSpec((1, block_rows, hidden), lambda j, i: (i, j, 0)),
            pl.BlockSpec((block_rows, g.shape[1]),
                         lambda j, i: (i * (seq // block_rows) + j, 0)),
            pl.BlockSpec((hidden, hidden), lambda j, i: (0, 0)),
            pl.BlockSpec((1, hidden), lambda j, i: (0, 0)),
            pl.BlockSpec((1, hidden), lambda j, i: (0, 0)),
            pl.BlockSpec((1, hidden), lambda j, i: (0, 0)),
            pl.BlockSpec((1, hidden), lambda j, i: (0, 0)),
        ],
        out_specs=pl.BlockSpec((1, block_rows, hidden), lambda j, i: (i, j, 0)),
        out_shape=jax.ShapeDtypeStruct((bsz, seq, hidden), jnp.float32),
        compiler_params=pltpu.CompilerParams(
            dimension_semantics=("arbitrary", "arbitrary"),
        ),
    )(mask2d, protein_embeddings, g, W, row(b), row(mask_embed),
      row(ln_gamma), row(ln_beta))

    return out
